# baseline (speedup 1.0000x reference)
import jax
import jax.numpy as jnp
from jax.experimental import pallas as pl
from jax.experimental.pallas import tpu as pltpu

_M = 4
_K = 1024
_D = 64
_N = 16384
_BLK = 512


def _drq_block_kernel(scale_ref, x_ref, cb_ref, codes_ref, loss_ref,
                      saug_scr, cts_scr, cn_scr):
    pid = pl.program_id(0)

    @pl.when(pid == 0)
    def _prep():
        c = cb_ref[...]
        kid = jax.lax.broadcasted_iota(
            jnp.int32, (_K, 128 - _D), 0).astype(jnp.float32)
        klane = jax.lax.broadcasted_iota(jnp.int32, (_K, 128 - _D), 1)
        hi = jnp.floor(kid / 256.0)
        const_cols = jnp.where(
            klane == 0, 1.0,
            jnp.where(klane == 1, hi,
                      jnp.where(klane == 2, kid - 256.0 * hi, 0.0)))
        for i in range(_M):
            s = scale_ref[i]
            ci = c * s
            saug_scr[i, :, :_D] = ci
            saug_scr[i, :, _D:] = const_cols
            cts_scr[i] = jnp.transpose(ci, (1, 0))
            cn_scr[i, :] = jnp.sum(ci * ci, axis=1)

    x0 = x_ref[...]
    r = x0
    qsoft = jnp.zeros_like(x0)
    qhard = jnp.zeros_like(x0)
    acc_sd = jnp.zeros_like(x0)
    acc_hd = jnp.zeros_like(x0)
    lane = jax.lax.broadcasted_iota(jnp.int32, (1, 128), 1)

    codes_cols = []
    for i in range(_M):
        saug = saug_scr[i]
        cn = cn_scr[i, :]
        rn = jnp.sum(r * r, axis=1, keepdims=True)
        dots2 = jax.lax.dot_general(
            r + r, cts_scr[i], (((1,), (0,)), ((), ())),
            preferred_element_type=jnp.float32)
        d = (rn - dots2) + cn[None, :]
        mn = jnp.min(d, axis=1, keepdims=True)
        e = jnp.exp(mn - d)
        mask = (d == mn).astype(jnp.float32)
        out1 = jax.lax.dot_general(
            e, saug, (((1,), (0,)), ((), ())),
            preferred_element_type=jnp.float32)
        out2 = jax.lax.dot_general(
            mask, saug, (((1,), (0,)), ((), ())),
            preferred_element_type=jnp.float32)
        soft = out1[:, :_D] / out1[:, _D:_D + 1]
        hard = out2[:, :_D]
        code_f = out2[:, _D + 1:_D + 2] * 256.0 + out2[:, _D + 2:_D + 3]
        r = r - hard
        qsoft = qsoft + soft
        qhard = qhard + hard
        dso = x0 - qsoft
        dha = x0 - qhard
        acc_sd = acc_sd + dso * dso
        acc_hd = acc_hd + dha * dha
        codes_cols.append(code_f.astype(jnp.int32))

    codes_ref[...] = jnp.concatenate(codes_cols, axis=1)

    djc = qsoft - qhard
    blk = 0.1 * jnp.sum(acc_sd) + jnp.sum(acc_hd) + 0.1 * jnp.sum(djc * djc)
    contrib = blk * (1.0 / (_N * _D))
    row = jnp.where(lane == 0, contrib, 0.0)

    @pl.when(pid == 0)
    def _init():
        loss_ref[...] = jnp.zeros_like(loss_ref)

    loss_ref[...] += row


def kernel(x, codebook, scale):
    nblk = _N // _BLK
    codes, loss_row = pl.pallas_call(
        _drq_block_kernel,
        grid=(nblk,),
        in_specs=[
            pl.BlockSpec(memory_space=pltpu.SMEM),
            pl.BlockSpec((_BLK, _D), lambda i: (i, 0)),
            pl.BlockSpec((_K, _D), lambda i: (0, 0)),
        ],
        out_specs=[
            pl.BlockSpec((_BLK, _M), lambda i: (i, 0)),
            pl.BlockSpec((1, 128), lambda i: (0, 0)),
        ],
        out_shape=[
            jax.ShapeDtypeStruct((_N, _M), jnp.int32),
            jax.ShapeDtypeStruct((1, 128), jnp.float32),
        ],
        scratch_shapes=[
            pltpu.VMEM((_M, _K, 128), jnp.float32),
            pltpu.VMEM((_M, _D, _K), jnp.float32),
            pltpu.VMEM((_M, _K), jnp.float32),
        ],
        compiler_params=pltpu.CompilerParams(
            dimension_semantics=("arbitrary",)),
    )(scale, x, codebook)
    return (codes, loss_row[0, 0])


# speedup vs baseline: 1.0184x; 1.0184x over previous
import jax
import jax.numpy as jnp
from jax.experimental import pallas as pl
from jax.experimental.pallas import tpu as pltpu

_M = 4
_K = 1024
_D = 64
_N = 16384
_BLK = 1024


def _drq_block_kernel(scale_ref, x_ref, cb_ref, codes_ref, loss_ref,
                      saug_scr, cts_scr, cn_scr):
    pid = pl.program_id(0)

    @pl.when(pid == 0)
    def _prep():
        c = cb_ref[...]
        kid = jax.lax.broadcasted_iota(
            jnp.int32, (_K, 128 - _D), 0).astype(jnp.float32)
        klane = jax.lax.broadcasted_iota(jnp.int32, (_K, 128 - _D), 1)
        hi = jnp.floor(kid / 256.0)
        const_cols = jnp.where(
            klane == 0, 1.0,
            jnp.where(klane == 1, hi,
                      jnp.where(klane == 2, kid - 256.0 * hi, 0.0)))
        for i in range(_M):
            s = scale_ref[i]
            ci = c * s
            saug_scr[i, :, :_D] = ci
            saug_scr[i, :, _D:] = const_cols
            cts_scr[i] = jnp.transpose(ci, (1, 0))
            cn_scr[i, :] = jnp.sum(ci * ci, axis=1)

    lane = jax.lax.broadcasted_iota(jnp.int32, (1, 128), 1)
    _H = _BLK // 2
    x0 = [x_ref[:_H], x_ref[_H:]]
    r = list(x0)
    qsoft = [jnp.zeros_like(x0[0]) for _ in range(2)]
    qhard = [jnp.zeros_like(x0[0]) for _ in range(2)]
    acc_sd = [jnp.zeros_like(x0[0]) for _ in range(2)]
    acc_hd = [jnp.zeros_like(x0[0]) for _ in range(2)]

    codes_cols = [[], []]
    for i in range(_M):
        saug = saug_scr[i]
        cn = cn_scr[i, :]
        for h in range(2):
            rn = jnp.sum(r[h] * r[h], axis=1, keepdims=True)
            dots2 = jax.lax.dot_general(
                r[h] + r[h], cts_scr[i], (((1,), (0,)), ((), ())),
                preferred_element_type=jnp.float32)
            d = (rn - dots2) + cn[None, :]
            mn = jnp.min(d, axis=1, keepdims=True)
            e = jnp.exp(mn - d)
            mask = (d == mn).astype(jnp.float32)
            out1 = jax.lax.dot_general(
                e, saug, (((1,), (0,)), ((), ())),
                preferred_element_type=jnp.float32)
            out2 = jax.lax.dot_general(
                mask, saug, (((1,), (0,)), ((), ())),
                preferred_element_type=jnp.float32)
            soft = out1[:, :_D] / out1[:, _D:_D + 1]
            hard = out2[:, :_D]
            code_f = (out2[:, _D + 1:_D + 2] * 256.0
                      + out2[:, _D + 2:_D + 3])
            r[h] = r[h] - hard
            qsoft[h] = qsoft[h] + soft
            qhard[h] = qhard[h] + hard
            dso = x0[h] - qsoft[h]
            dha = x0[h] - qhard[h]
            acc_sd[h] = acc_sd[h] + dso * dso
            acc_hd[h] = acc_hd[h] + dha * dha
            codes_cols[h].append(code_f.astype(jnp.int32))

    codes_ref[:_H, :] = jnp.concatenate(codes_cols[0], axis=1)
    codes_ref[_H:, :] = jnp.concatenate(codes_cols[1], axis=1)

    blk = jnp.float32(0.0)
    for h in range(2):
        djc = qsoft[h] - qhard[h]
        blk = blk + (0.1 * jnp.sum(acc_sd[h]) + jnp.sum(acc_hd[h])
                     + 0.1 * jnp.sum(djc * djc))
    contrib = blk * (1.0 / (_N * _D))
    row = jnp.where(lane == 0, contrib, 0.0)

    @pl.when(pid == 0)
    def _init():
        loss_ref[...] = jnp.zeros_like(loss_ref)

    loss_ref[...] += row


def kernel(x, codebook, scale):
    nblk = _N // _BLK
    codes, loss_row = pl.pallas_call(
        _drq_block_kernel,
        grid=(nblk,),
        in_specs=[
            pl.BlockSpec(memory_space=pltpu.SMEM),
            pl.BlockSpec((_BLK, _D), lambda i: (i, 0)),
            pl.BlockSpec((_K, _D), lambda i: (0, 0)),
        ],
        out_specs=[
            pl.BlockSpec((_BLK, _M), lambda i: (i, 0)),
            pl.BlockSpec((1, 128), lambda i: (0, 0)),
        ],
        out_shape=[
            jax.ShapeDtypeStruct((_N, _M), jnp.int32),
            jax.ShapeDtypeStruct((1, 128), jnp.float32),
        ],
        scratch_shapes=[
            pltpu.VMEM((_M, _K, 128), jnp.float32),
            pltpu.VMEM((_M, _D, _K), jnp.float32),
            pltpu.VMEM((_M, _K), jnp.float32),
        ],
        compiler_params=pltpu.CompilerParams(
            dimension_semantics=("arbitrary",)),
    )(scale, x, codebook)
    return (codes, loss_row[0, 0])
